# baseline (speedup 1.0000x reference)
import functools

import jax
import jax.numpy as jnp
from jax import lax
from jax.experimental import pallas as pl
from jax.experimental.pallas import tpu as pltpu
from jax.experimental.pallas import tpu_sc as plsc

B, S, D = 4, 2048, 1024
BS = 2
NUM_CORES = 2
NUM_SUBCORES = 16
NW = NUM_CORES * NUM_SUBCORES
POS_PER_W = S // NW
R = 16
NLB = POS_PER_W // R
NSTEP = NLB * BS
NR = 4
L = 3


def _build(interpret=False):
  mesh = plsc.VectorSubcoreMesh(
      core_axis_name="c", subcore_axis_name="s",
      num_cores=NUM_CORES, num_subcores=NUM_SUBCORES)

  scratch = (
      [pltpu.VMEM((R, D), jnp.float32) for _ in range(NR)]
      + [pltpu.VMEM((R, D), jnp.float32) for _ in range(2)]
      + [pltpu.SemaphoreType.DMA for _ in range(2 * NR + 2)]
  )

  @functools.partial(
      pl.kernel,
      out_type=jax.ShapeDtypeStruct((BS, S, D), jnp.float32),
      mesh=mesh,
      scratch_types=scratch,
      interpret=interpret,
  )
  def sc_add(x_hbm, lut_hbm, out_hbm, *scr):
    xbuf = scr[:NR]
    lbuf = scr[NR:NR + 2]
    sem_ld = scr[NR + 2:NR + 2 + NR]
    sem_st = scr[NR + 2 + NR:NR + 2 + 2 * NR]
    sem_lut = scr[NR + 2 + 2 * NR:]

    w = lax.axis_index("s") * NUM_CORES + lax.axis_index("c")
    pos0 = w * POS_PER_W

    loads, lloads, stores = {}, {}, {}
    waited = set()

    def issue_load(s):
      lb, b = divmod(s, BS)
      r = s % NR
      loads[s] = pltpu.async_copy(
          x_hbm.at[b, pl.ds(pos0 + lb * R, R), :], xbuf[r], sem_ld[r])

    def issue_lut(lb):
      lloads[lb] = pltpu.async_copy(
          lut_hbm.at[pl.ds(pos0 + lb * R, R), :], lbuf[lb % 2],
          sem_lut[lb % 2])

    issue_lut(0)
    if NLB > 1:
      issue_lut(1)
    for s in range(min(L, NSTEP)):
      issue_load(s)

    for s in range(NSTEP):
      lb, b = divmod(s, BS)
      ss = s + L
      if ss < NSTEP:
        if ss - NR >= 0:
          stores[ss - NR].wait()
          waited.add(ss - NR)
        issue_load(ss)
      if b == 0:
        lloads[lb].wait()
      r = s % NR
      loads[s].wait()
      xb, lbf = xbuf[r], lbuf[lb % 2]

      @plsc.parallel_loop(0, R * D, 16, unroll=8)
      def add_body(i):
        row = i >> 10
        col = pl.multiple_of(i & (D - 1), 16)
        plsc.addupdate(xb.at[row, pl.ds(col, 16)], lbf[row, pl.ds(col, 16)])

      stores[s] = pltpu.async_copy(
          xb, out_hbm.at[b, pl.ds(pos0 + lb * R, R), :], sem_st[r])
      if b == BS - 1 and lb + 2 < NLB:
        issue_lut(lb + 2)

    for s in range(NSTEP):
      if s not in waited:
        stores[s].wait()

  return sc_add


_sc_add = _build()

RB = 256


def _tc_body(x_ref, lut_ref, o_ref):
  o_ref[...] = x_ref[...] + lut_ref[...][None]


_tc_add = pl.pallas_call(
    _tc_body,
    grid=(B - BS, S // RB),
    in_specs=[
        pl.BlockSpec((1, RB, D), lambda b, i: (b + BS, i, 0)),
        pl.BlockSpec((RB, D), lambda b, i: (i, 0)),
    ],
    out_specs=pl.BlockSpec((1, RB, D), lambda b, i: (b, i, 0)),
    out_shape=jax.ShapeDtypeStruct((B - BS, S, D), jnp.float32),
)


@jax.jit
def kernel(x, lut):
  sc_out = _sc_add(x, lut)
  tc_out = _tc_add(x, lut)
  return jnp.concatenate([sc_out, tc_out], axis=0)


# speedup vs baseline: 1.4047x; 1.4047x over previous
import functools

import jax
import jax.numpy as jnp
from jax import lax
from jax.experimental import pallas as pl
from jax.experimental.pallas import tpu as pltpu
from jax.experimental.pallas import tpu_sc as plsc

B, S, D = 4, 2048, 1024
NUM_CORES = 2
NUM_SUBCORES = 16
NW = NUM_CORES * NUM_SUBCORES
POS_PER_W = S // NW
R = 8
NLB = POS_PER_W // R
NG = 3


def _build(interpret=False):
  mesh = plsc.VectorSubcoreMesh(
      core_axis_name="c", subcore_axis_name="s",
      num_cores=NUM_CORES, num_subcores=NUM_SUBCORES)

  scratch = (
      [pltpu.VMEM((R, D), jnp.float32) for _ in range(NG * B)]
      + [pltpu.VMEM((R, D), jnp.float32) for _ in range(2)]
      + [pltpu.SemaphoreType.DMA for _ in range(2 * NG * B + 2)]
  )

  @functools.partial(
      pl.kernel,
      out_type=jax.ShapeDtypeStruct((B, S, D), jnp.float32),
      mesh=mesh,
      scratch_types=scratch,
      interpret=interpret,
  )
  def sc_add(x_hbm, lut_hbm, out_hbm, *scr):
    xbuf = scr[:NG * B]
    lbuf = scr[NG * B:NG * B + 2]
    sem_ld = scr[NG * B + 2:2 * NG * B + 2]
    sem_st = scr[2 * NG * B + 2:3 * NG * B + 2]
    sem_lut = scr[3 * NG * B + 2:]

    w = lax.axis_index("s") * NUM_CORES + lax.axis_index("c")
    pos0 = w * POS_PER_W

    loads, lloads, stores = {}, {}, {}
    waited = set()

    def issue_loads(lb):
      g = lb % NG
      loads[lb] = [
          pltpu.async_copy(x_hbm.at[b, pl.ds(pos0 + lb * R, R), :],
                           xbuf[g * B + b], sem_ld[g * B + b])
          for b in range(B)]

    def issue_lut(lb):
      lloads[lb] = pltpu.async_copy(
          lut_hbm.at[pl.ds(pos0 + lb * R, R), :], lbuf[lb % 2],
          sem_lut[lb % 2])

    issue_lut(0)
    issue_lut(1)
    issue_loads(0)
    issue_loads(1)

    for lb in range(NLB):
      g = lb % NG
      lloads[lb].wait()
      for c in loads[lb]:
        c.wait()
      lbf = lbuf[lb % 2]
      xbs = [xbuf[g * B + b] for b in range(B)]

      @plsc.parallel_loop(0, R * D, 16, unroll=4)
      def add_body(i):
        row = i >> 10
        col = pl.multiple_of(i & (D - 1), 16)
        v = lbf[row, pl.ds(col, 16)]
        for xb in xbs:
          plsc.addupdate(xb.at[row, pl.ds(col, 16)], v)

      stores[lb] = [
          pltpu.async_copy(xbs[b], out_hbm.at[b, pl.ds(pos0 + lb * R, R), :],
                           sem_st[g * B + b])
          for b in range(B)]

      nb = lb + 2
      if nb < NLB:
        if nb - NG >= 0:
          for c in stores[nb - NG]:
            c.wait()
          waited.add(nb - NG)
        issue_loads(nb)
        issue_lut(nb)

    for lb in range(NLB):
      if lb not in waited:
        for c in stores[lb]:
          c.wait()

  return sc_add


_sc_add = _build()


@jax.jit
def kernel(x, lut):
  return _sc_add(x, lut)
